# baseline (speedup 1.0000x reference)
import functools

import jax
import jax.numpy as jnp
from jax import lax
from jax.experimental import pallas as pl
from jax.experimental.pallas import tpu as pltpu
from jax.experimental.pallas import tpu_sc as plsc

_L = 16
_NC, _NS = 2, 16
_NW = _NC * _NS
_UNROLL = 8
_NCHUNK = 5

_N_TC = 614400
_BS_TC = 122880


def _sc_run_factory(n, n_sc, k):
    off = n - n_sc
    q = _UNROLL * _NCHUNK
    nv = -(-(n_sc // _L) // _NW)
    nv = -(-nv // q) * q
    ch = nv * _L
    cnv = nv // _NCHUNK
    cch = cnv * _L
    kmax = k - 1

    mesh = plsc.VectorSubcoreMesh(core_axis_name="c", subcore_axis_name="s")

    @functools.partial(
        pl.kernel,
        out_type=jax.ShapeDtypeStruct((n_sc,), jnp.float32),
        mesh=mesh,
        compiler_params=pltpu.CompilerParams(needs_layout_passes=False),
        scratch_types=[
            pltpu.VMEM((cch,), jnp.float32),
            pltpu.VMEM((cch,), jnp.float32),
            pltpu.VMEM((cch,), jnp.int32),
            pltpu.VMEM((cch,), jnp.int32),
            pltpu.VMEM((cch,), jnp.float32),
            pltpu.VMEM((cch,), jnp.float32),
            pltpu.VMEM((3 * _L,), jnp.float32),
            pltpu.SemaphoreType.DMA,
            pltpu.SemaphoreType.DMA,
            pltpu.SemaphoreType.DMA,
            pltpu.SemaphoreType.DMA,
            pltpu.SemaphoreType.DMA,
            pltpu.SemaphoreType.DMA,
        ],
    )
    def run(logits_hbm, counts_hbm, params_hbm, out_hbm,
            lg0, lg1, ct0, ct1, o0, o1, par_v,
            slg0, slg1, sct0, sct1, sout0, sout1):
        lg_b = (lg0, lg1)
        ct_b = (ct0, ct1)
        out_b = (o0, o1)
        slg = (slg0, slg1)
        sct = (sct0, sct1)
        sout = (sout0, sout1)
        wid = lax.axis_index("s") * _NC + lax.axis_index("c")
        base = jnp.minimum(wid * ch, n_sc - ch)
        pltpu.sync_copy(params_hbm, par_v)
        tabr = par_v.at[pl.ds(0, _L)]
        pmv = par_v[pl.ds(_L, _L)]
        p2mv = par_v[pl.ds(2 * _L, _L)]

        def start_in(j):
            b = j % 2
            src = off + base + j * cch
            hl = pltpu.async_copy(
                logits_hbm.at[pl.ds(src, cch)], lg_b[b], slg[b])
            hc = pltpu.async_copy(
                counts_hbm.at[pl.ds(src, cch)], ct_b[b], sct[b])
            return hl, hc

        hin = [None] * _NCHUNK
        hout = [None] * _NCHUNK
        hin[0] = start_in(0)
        for j in range(_NCHUNK):
            if j + 1 < _NCHUNK:
                hin[j + 1] = start_in(j + 1)
            hin[j][0].wait()
            hin[j][1].wait()
            if j >= 2:
                hout[j - 2].wait()
            b = j % 2
            lgb, ctb, outb = lg_b[b], ct_b[b], out_b[b]

            @plsc.parallel_loop(0, cnv, 1, unroll=_UNROLL)
            def body(i):
                x = lgb[pl.ds(i * _L, _L)]
                ci = jnp.minimum(ctb[pl.ds(i * _L, _L)], kmax)
                c = plsc.load_gather(tabr, [ci])
                e = jnp.exp(x * c)
                outb[pl.ds(i * _L, _L)] = pmv - p2mv / (e + 1.0)

            hout[j] = pltpu.async_copy(
                outb, out_hbm.at[pl.ds(base + j * cch, cch)], sout[b])
        hout[_NCHUNK - 2].wait()
        hout[_NCHUNK - 1].wait()

    return run


def _tc_body(par_ref, lg_ref, ct_ref, o_ref):
    x = lg_ref[...]
    idx = jnp.minimum(ct_ref[...], 10)
    g = jnp.zeros_like(x) + par_ref[0]
    for t in range(1, 11):
        g = jnp.where(idx >= t, par_ref[t], g)
    m = par_ref[16]
    o_ref[...] = m * jnp.tanh((0.5 * x) * g)


def kernel(logits, alt_counts, confidence, max_logit):
    n = logits.shape[0]
    k = confidence.shape[0]
    n_tc = _N_TC
    n_sc = n - n_tc

    m = max_logit.astype(jnp.float32)
    i48 = lax.iota(jnp.int32, 3 * _L)
    c48 = jnp.take(confidence, jnp.clip(i48, 0, k - 1), mode="clip") * (2.0 / m)
    params = jnp.where(
        i48 < k, c48, jnp.where(i48 < _L, 0.0, jnp.where(i48 < 2 * _L, m, 2.0 * m))
    )

    sc_run = _sc_run_factory(n, n_sc, k)
    sc_out = sc_run(logits, alt_counts, params)

    grid = n_tc // _BS_TC
    tc_full = pl.pallas_call(
        _tc_body,
        grid=(grid,),
        in_specs=[
            pl.BlockSpec((48,), lambda i: (0,)),
            pl.BlockSpec((_BS_TC,), lambda i: (i,)),
            pl.BlockSpec((_BS_TC,), lambda i: (i,)),
        ],
        out_specs=pl.BlockSpec((_BS_TC,), lambda i: (i,)),
        out_shape=jax.ShapeDtypeStruct((n,), jnp.float32),
    )(params, logits, alt_counts)

    return lax.dynamic_update_slice(tc_full, sc_out, (n_tc,))


# speedup vs baseline: 1.0451x; 1.0451x over previous
import functools

import jax
import jax.numpy as jnp
from jax import lax
from jax.experimental import pallas as pl
from jax.experimental.pallas import tpu as pltpu
from jax.experimental.pallas import tpu_sc as plsc

_L = 16
_NC, _NS = 2, 16
_NW = _NC * _NS
_UNROLL = 8
_NCHUNK = 5

_N_TC = 737280
_BS_TC = 122880


def _sc_run_factory(n, n_sc, k):
    off = n - n_sc
    q = _UNROLL * _NCHUNK
    nv = -(-(n_sc // _L) // _NW)
    nv = -(-nv // q) * q
    ch = nv * _L
    cnv = nv // _NCHUNK
    cch = cnv * _L
    kmax = k - 1

    mesh = plsc.VectorSubcoreMesh(core_axis_name="c", subcore_axis_name="s")

    @functools.partial(
        pl.kernel,
        out_type=jax.ShapeDtypeStruct((n_sc,), jnp.float32),
        mesh=mesh,
        compiler_params=pltpu.CompilerParams(needs_layout_passes=False),
        scratch_types=[
            pltpu.VMEM((cch,), jnp.float32),
            pltpu.VMEM((cch,), jnp.float32),
            pltpu.VMEM((cch,), jnp.int32),
            pltpu.VMEM((cch,), jnp.int32),
            pltpu.VMEM((cch,), jnp.float32),
            pltpu.VMEM((cch,), jnp.float32),
            pltpu.VMEM((3 * _L,), jnp.float32),
            pltpu.SemaphoreType.DMA,
            pltpu.SemaphoreType.DMA,
            pltpu.SemaphoreType.DMA,
            pltpu.SemaphoreType.DMA,
            pltpu.SemaphoreType.DMA,
            pltpu.SemaphoreType.DMA,
        ],
    )
    def run(logits_hbm, counts_hbm, params_hbm, out_hbm,
            lg0, lg1, ct0, ct1, o0, o1, par_v,
            slg0, slg1, sct0, sct1, sout0, sout1):
        lg_b = (lg0, lg1)
        ct_b = (ct0, ct1)
        out_b = (o0, o1)
        slg = (slg0, slg1)
        sct = (sct0, sct1)
        sout = (sout0, sout1)
        wid = lax.axis_index("s") * _NC + lax.axis_index("c")
        base = jnp.minimum(wid * ch, n_sc - ch)
        pltpu.sync_copy(params_hbm, par_v)
        tabr = par_v.at[pl.ds(0, _L)]
        pmv = par_v[pl.ds(_L, _L)]
        p2mv = par_v[pl.ds(2 * _L, _L)]

        def start_in(j):
            b = j % 2
            src = off + base + j * cch
            hl = pltpu.async_copy(
                logits_hbm.at[pl.ds(src, cch)], lg_b[b], slg[b])
            hc = pltpu.async_copy(
                counts_hbm.at[pl.ds(src, cch)], ct_b[b], sct[b])
            return hl, hc

        hin = [None] * _NCHUNK
        hout = [None] * _NCHUNK
        hin[0] = start_in(0)
        for j in range(_NCHUNK):
            if j + 1 < _NCHUNK:
                hin[j + 1] = start_in(j + 1)
            hin[j][0].wait()
            hin[j][1].wait()
            if j >= 2:
                hout[j - 2].wait()
            b = j % 2
            lgb, ctb, outb = lg_b[b], ct_b[b], out_b[b]

            @plsc.parallel_loop(0, cnv, 1, unroll=_UNROLL)
            def body(i):
                x = lgb[pl.ds(i * _L, _L)]
                ci = jnp.minimum(ctb[pl.ds(i * _L, _L)], kmax)
                c = plsc.load_gather(tabr, [ci])
                e = jnp.exp(x * c)
                outb[pl.ds(i * _L, _L)] = pmv - p2mv / (e + 1.0)

            hout[j] = pltpu.async_copy(
                outb, out_hbm.at[pl.ds(base + j * cch, cch)], sout[b])
        hout[_NCHUNK - 2].wait()
        hout[_NCHUNK - 1].wait()

    return run


def _tc_body(par_ref, lg_ref, ct_ref, o_ref):
    x = lg_ref[...]
    idx = jnp.minimum(ct_ref[...], 10)
    g = jnp.zeros_like(x) + par_ref[0]
    for t in range(1, 11):
        g = jnp.where(idx >= t, par_ref[t], g)
    m = par_ref[16]
    o_ref[...] = m * jnp.tanh((0.5 * x) * g)


def kernel(logits, alt_counts, confidence, max_logit):
    n = logits.shape[0]
    k = confidence.shape[0]
    n_tc = _N_TC
    n_sc = n - n_tc

    m = max_logit.astype(jnp.float32)
    tab = jnp.zeros((_L,), jnp.float32).at[:k].set(confidence * (2.0 / m))
    params = jnp.concatenate(
        [tab, jnp.full((_L,), m, jnp.float32), jnp.full((_L,), 2.0 * m, jnp.float32)]
    )

    sc_run = _sc_run_factory(n, n_sc, k)
    sc_out = sc_run(logits, alt_counts, params)

    grid = n_tc // _BS_TC
    tc_full = pl.pallas_call(
        _tc_body,
        grid=(grid,),
        in_specs=[
            pl.BlockSpec((48,), lambda i: (0,)),
            pl.BlockSpec((_BS_TC,), lambda i: (i,)),
            pl.BlockSpec((_BS_TC,), lambda i: (i,)),
        ],
        out_specs=pl.BlockSpec((_BS_TC,), lambda i: (i,)),
        out_shape=jax.ShapeDtypeStruct((n,), jnp.float32),
    )(params, logits, alt_counts)

    return lax.dynamic_update_slice(tc_full, sc_out, (n_tc,))


# speedup vs baseline: 1.0681x; 1.0221x over previous
import functools

import jax
import jax.numpy as jnp
from jax import lax
from jax.experimental import pallas as pl
from jax.experimental.pallas import tpu as pltpu
from jax.experimental.pallas import tpu_sc as plsc

_L = 16
_NC, _NS = 2, 16
_NW = _NC * _NS
_UNROLL = 8
_NCHUNK = 8
_NBUF = 3


def kernel(logits, alt_counts, confidence, max_logit):
    n = logits.shape[0]
    k = confidence.shape[0]
    q = _UNROLL * _NCHUNK
    nv = -(-(n // _L) // _NW)
    nv = -(-nv // q) * q
    ch = nv * _L
    cnv = nv // _NCHUNK
    cch = cnv * _L
    kmax = k - 1

    m = max_logit.astype(jnp.float32)
    tab = jnp.zeros((_L,), jnp.float32).at[:k].set(confidence * (2.0 / m))
    params = jnp.concatenate(
        [tab, jnp.full((_L,), m, jnp.float32), jnp.full((_L,), 2.0 * m, jnp.float32)]
    )

    mesh = plsc.VectorSubcoreMesh(core_axis_name="c", subcore_axis_name="s")

    vmem = pltpu.VMEM
    sem = pltpu.SemaphoreType.DMA
    scratch = (
        [vmem((cch,), jnp.float32) for _ in range(_NBUF)]
        + [vmem((cch,), jnp.int32) for _ in range(_NBUF)]
        + [vmem((cch,), jnp.float32) for _ in range(_NBUF)]
        + [vmem((3 * _L,), jnp.float32)]
        + [sem] * (3 * _NBUF + 1)
    )

    @functools.partial(
        pl.kernel,
        out_type=jax.ShapeDtypeStruct((n,), jnp.float32),
        mesh=mesh,
        compiler_params=pltpu.CompilerParams(needs_layout_passes=False),
        scratch_types=scratch,
    )
    def run(logits_hbm, counts_hbm, params_hbm, out_hbm, *bufs):
        lg_b = bufs[0:_NBUF]
        ct_b = bufs[_NBUF:2 * _NBUF]
        out_b = bufs[2 * _NBUF:3 * _NBUF]
        par_v = bufs[3 * _NBUF]
        slg = bufs[3 * _NBUF + 1:3 * _NBUF + 1 + _NBUF]
        sct = bufs[3 * _NBUF + 1 + _NBUF:3 * _NBUF + 1 + 2 * _NBUF]
        sout = bufs[3 * _NBUF + 1 + 2 * _NBUF:3 * _NBUF + 1 + 3 * _NBUF]
        spar = bufs[3 * _NBUF + 1 + 3 * _NBUF]
        wid = lax.axis_index("s") * _NC + lax.axis_index("c")
        base = jnp.minimum(wid * ch, n - ch)

        def start_in(j):
            b = j % _NBUF
            hl = pltpu.async_copy(
                logits_hbm.at[pl.ds(base + j * cch, cch)], lg_b[b], slg[b])
            hc = pltpu.async_copy(
                counts_hbm.at[pl.ds(base + j * cch, cch)], ct_b[b], sct[b])
            return hl, hc

        hin = [None] * _NCHUNK
        hout = [None] * _NCHUNK
        hin[0] = start_in(0)
        hpar = pltpu.async_copy(params_hbm, par_v, spar)
        hin[1] = start_in(1)
        hpar.wait()
        tabr = par_v.at[pl.ds(0, _L)]
        pmv = par_v[pl.ds(_L, _L)]
        p2mv = par_v[pl.ds(2 * _L, _L)]

        for j in range(_NCHUNK):
            if j + 2 < _NCHUNK:
                hin[j + 2] = start_in(j + 2)
            hin[j][0].wait()
            hin[j][1].wait()
            if j >= _NBUF:
                hout[j - _NBUF].wait()
            b = j % _NBUF
            lgb, ctb, outb = lg_b[b], ct_b[b], out_b[b]

            @plsc.parallel_loop(0, cnv, 1, unroll=_UNROLL)
            def body(i):
                x = lgb[pl.ds(i * _L, _L)]
                ci = jnp.minimum(ctb[pl.ds(i * _L, _L)], kmax)
                c = plsc.load_gather(tabr, [ci])
                e = jnp.exp(x * c)
                outb[pl.ds(i * _L, _L)] = pmv - p2mv / (e + 1.0)

            hout[j] = pltpu.async_copy(
                outb, out_hbm.at[pl.ds(base + j * cch, cch)], sout[b])
        for j in range(max(0, _NCHUNK - _NBUF), _NCHUNK):
            hout[j].wait()

    return run(logits, alt_counts, params)
